# baseline (speedup 1.0000x reference)
import functools

import jax
import jax.numpy as jnp
from jax import lax
from jax.experimental import pallas as pl
from jax.experimental.pallas import tpu as pltpu
from jax.experimental.pallas import tpu_sc as plsc

E = 8
D = 1024
FF = 2048
T = 2048
BLK = 128
NB = T // BLK
SP = NB + E - 1
NBP = NB + E
TP = NBP * BLK
RBLK = 128
RNB = T // RBLK
SM = 32
NWORKERS = 32
CHUNK = T // NWORKERS


def _dg(a, b, ca, cb):
    return lax.dot_general(a, b, (((ca,), (cb,)), ((), ())),
                           preferred_element_type=jnp.float32)


def _routing_body(ids_ref, pos_ref, meta_ref):
    e_row = lax.broadcasted_iota(jnp.int32, (1, E), 1)
    tri_r = lax.broadcasted_iota(jnp.int32, (RBLK, RBLK), 0)
    tri_c = lax.broadcasted_iota(jnp.int32, (RBLK, RBLK), 1)
    tri = (tri_r >= tri_c).astype(jnp.float32)

    def pass1(c, running):
        ids_c = ids_ref[pl.ds(c * RBLK, RBLK), :]
        oh = (ids_c == e_row).astype(jnp.float32)
        ranks = _dg(tri, oh, 1, 0) + running
        grank = jnp.sum(oh * (ranks - 1.0), axis=1, keepdims=True)
        pos_ref[pl.ds(c * RBLK, RBLK), :] = grank.astype(jnp.int32)
        return ranks[RBLK - 1:RBLK, :]

    counts = lax.fori_loop(0, RNB, pass1, jnp.zeros((1, E), jnp.float32))

    lt8_r = lax.broadcasted_iota(jnp.int32, (E, E), 0)
    lt8_c = lax.broadcasted_iota(jnp.int32, (E, E), 1)
    lt8 = (lt8_r < lt8_c).astype(jnp.float32)
    nblk = jnp.floor((counts + float(BLK - 1)) * (1.0 / BLK))
    ps = _dg(nblk, lt8, 1, 0)
    pad_off = ps * float(BLK)

    def pass2(c, carry):
        ids_c = ids_ref[pl.ds(c * RBLK, RBLK), :]
        oh = (ids_c == e_row).astype(jnp.float32)
        add = jnp.sum(oh * pad_off, axis=1, keepdims=True).astype(jnp.int32)
        pos_ref[pl.ds(c * RBLK, RBLK), :] = (
            pos_ref[pl.ds(c * RBLK, RBLK), :] + add)
        return carry

    lax.fori_loop(0, RNB, pass2, 0)

    total = ps[0, E - 1] + nblk[0, E - 1]
    s_f = lax.broadcasted_iota(jnp.int32, (SM, 1), 0).astype(jnp.float32)
    e_col = jnp.sum((ps <= s_f).astype(jnp.float32), axis=1,
                    keepdims=True) - 1.0
    e_col = jnp.clip(e_col, 0.0, float(E - 1))
    valid = (s_f < total).astype(jnp.float32)
    xb = jnp.minimum(s_f, total - 1.0)

    prev_e = jnp.concatenate(
        [jnp.full((1, 1), -1.0, jnp.float32), e_col[:SM - 1]], axis=0)
    chg = ((e_col != prev_e) & (valid > 0)).astype(jnp.float32)
    idm_r = lax.broadcasted_iota(jnp.int32, (SM, SM), 0)
    idm_c = lax.broadcasted_iota(jnp.int32, (SM, SM), 1)
    idm = (idm_r == idm_c).astype(jnp.float32)
    trim = (idm_r >= idm_c).astype(jnp.float32)
    dix = _dg(trim, chg, 1, 0) - 1.0
    slot = dix - 2.0 * jnp.floor(dix * 0.5)
    islot = 1.0 - slot
    dix_row = _dg(dix, idm, 0, 0)
    chg_row = _dg(chg, idm, 0, 0)
    nxt = (dix_row == dix + 1.0).astype(jnp.float32) * chg_row
    nxe = _dg(nxt, e_col, 1, 0)
    nxv = _dg(nxt, jnp.ones((SM, 1), jnp.float32), 1, 0)
    issue = chg * (nxv > 0).astype(jnp.float32)

    meta = jnp.concatenate(
        [xb, e_col, valid, chg, slot, issue, islot, nxe], axis=1)
    meta_ref[...] = meta.astype(jnp.int32)


def _route(ids):
    return pl.pallas_call(
        _routing_body,
        out_shape=(jax.ShapeDtypeStruct((T, 1), jnp.int32),
                   jax.ShapeDtypeStruct((SM, 8), jnp.int32)),
    )(ids)


@functools.lru_cache(maxsize=None)
def _sc_kernels():
    mesh = plsc.VectorSubcoreMesh(core_axis_name="c", subcore_axis_name="s")
    scratch = [pltpu.VMEM((CHUNK,), jnp.int32),
               pltpu.VMEM((CHUNK, D), jnp.float32),
               pltpu.SemaphoreType.DMA]

    @functools.partial(
        pl.kernel,
        out_type=jax.ShapeDtypeStruct((TP, D), jnp.float32),
        mesh=mesh, scratch_types=scratch)
    def dispatch(x_hbm, pos_hbm, out_hbm, idx_v, rows_v, sem):
        wid = lax.axis_index("s") * 2 + lax.axis_index("c")
        base = wid * CHUNK
        pltpu.sync_copy(pos_hbm.at[pl.ds(base, CHUNK)], idx_v)
        pltpu.sync_copy(x_hbm.at[pl.ds(base, CHUNK)], rows_v)
        pltpu.async_copy(rows_v, out_hbm.at[idx_v], sem).wait()

    @functools.partial(
        pl.kernel,
        out_type=jax.ShapeDtypeStruct((T, D), jnp.float32),
        mesh=mesh, scratch_types=scratch)
    def combine(y_hbm, pos_hbm, out_hbm, idx_v, rows_v, sem):
        wid = lax.axis_index("s") * 2 + lax.axis_index("c")
        base = wid * CHUNK
        pltpu.sync_copy(pos_hbm.at[pl.ds(base, CHUNK)], idx_v)
        pltpu.async_copy(y_hbm.at[idx_v], rows_v, sem).wait()
        pltpu.sync_copy(rows_v, out_hbm.at[pl.ds(base, CHUNK)])

    return dispatch, combine


def _mlp_body(meta_ref, x_ref, gw_hbm, uw_hbm, dw_hbm, o_ref,
              gw_v, uw_v, dw_v, sgw, suw, sdw):
    s = pl.program_id(0)
    ex = meta_ref[s, 1]
    va = meta_ref[s, 2]
    chg = meta_ref[s, 3]
    slot = meta_ref[s, 4]
    issue = meta_ref[s, 5]
    islot = meta_ref[s, 6]
    nxe = meta_ref[s, 7]

    def _start(e, k):
        pltpu.make_async_copy(gw_hbm.at[e], gw_v.at[k], sgw.at[k]).start()
        pltpu.make_async_copy(uw_hbm.at[e], uw_v.at[k], suw.at[k]).start()
        pltpu.make_async_copy(dw_hbm.at[e], dw_v.at[k], sdw.at[k]).start()

    @pl.when(s == 0)
    def _():
        _start(ex, slot)

    @pl.when(issue > 0)
    def _():
        _start(nxe, islot)

    @pl.when(chg > 0)
    def _():
        pltpu.make_async_copy(gw_hbm.at[0], gw_v.at[slot], sgw.at[slot]).wait()
        pltpu.make_async_copy(uw_hbm.at[0], uw_v.at[slot], suw.at[slot]).wait()
        pltpu.make_async_copy(dw_hbm.at[0], dw_v.at[slot], sdw.at[slot]).wait()

    @pl.when(va > 0)
    def _():
        x = x_ref[...]
        g = _dg(x, gw_v[slot], 1, 1)
        u = _dg(x, uw_v[slot], 1, 1)
        h = (g * jax.nn.sigmoid(g)) * u
        o_ref[...] = _dg(h, dw_v[slot], 1, 1)


def _mlp(meta, sorted_x, gate_w, up_w, down_w):
    grid_spec = pltpu.PrefetchScalarGridSpec(
        num_scalar_prefetch=1,
        grid=(SP,),
        in_specs=[
            pl.BlockSpec((BLK, D), lambda s, m: (m[s, 0], 0)),
            pl.BlockSpec(memory_space=pl.ANY),
            pl.BlockSpec(memory_space=pl.ANY),
            pl.BlockSpec(memory_space=pl.ANY),
        ],
        out_specs=pl.BlockSpec((BLK, D), lambda s, m: (m[s, 0], 0)),
        scratch_shapes=[
            pltpu.VMEM((2, FF, D), jnp.float32),
            pltpu.VMEM((2, FF, D), jnp.float32),
            pltpu.VMEM((2, D, FF), jnp.float32),
            pltpu.SemaphoreType.DMA((2,)),
            pltpu.SemaphoreType.DMA((2,)),
            pltpu.SemaphoreType.DMA((2,)),
        ],
    )
    return pl.pallas_call(
        _mlp_body,
        grid_spec=grid_spec,
        out_shape=jax.ShapeDtypeStruct((TP, D), jnp.float32),
    )(meta, sorted_x, gate_w, up_w, down_w)


def _scale_body(y_ref, w_ref, o_ref):
    o_ref[...] = y_ref[...] * w_ref[...]


def _scale(y, w):
    return pl.pallas_call(
        _scale_body,
        grid=(NB,),
        in_specs=[pl.BlockSpec((BLK, D), lambda i: (i, 0)),
                  pl.BlockSpec((BLK, 1), lambda i: (i, 0))],
        out_specs=pl.BlockSpec((BLK, D), lambda i: (i, 0)),
        out_shape=jax.ShapeDtypeStruct((T, D), jnp.float32),
    )(y, w)


def kernel(x, topk_ids, topk_weight, gate_w, up_w, down_w):
    ids = topk_ids.reshape(T, 1).astype(jnp.int32)
    pos2d, meta = _route(ids)
    pos = pos2d.reshape(T)

    dispatch, combine = _sc_kernels()
    sorted_x = dispatch(x, pos)
    sorted_y = _mlp(meta, sorted_x, gate_w, up_w, down_w)
    y_tok = combine(sorted_y, pos)
    return _scale(y_tok, topk_weight.reshape(T, 1).astype(jnp.float32))


# speedup vs baseline: 1.2747x; 1.2747x over previous
import functools

import jax
import jax.numpy as jnp
from jax import lax
from jax.experimental import pallas as pl
from jax.experimental.pallas import tpu as pltpu
from jax.experimental.pallas import tpu_sc as plsc

E = 8
D = 1024
FF = 2048
T = 2048
BLK = 256
NB = T // BLK
SP = NB + E - 1
NBP = NB + E
TP = NBP * BLK
RBLK = 128
RNB = T // RBLK
SM = 32
NWORKERS = 32
CHUNK = T // NWORKERS


def _dg(a, b, ca, cb):
    return lax.dot_general(a, b, (((ca,), (cb,)), ((), ())),
                           preferred_element_type=jnp.float32)


def _routing_body(ids_ref, pos_ref, meta_ref):
    e_row = lax.broadcasted_iota(jnp.int32, (1, E), 1)
    tri_r = lax.broadcasted_iota(jnp.int32, (RBLK, RBLK), 0)
    tri_c = lax.broadcasted_iota(jnp.int32, (RBLK, RBLK), 1)
    tri = (tri_r >= tri_c).astype(jnp.float32)

    def pass1(c, running):
        ids_c = ids_ref[pl.ds(c * RBLK, RBLK), :]
        oh = (ids_c == e_row).astype(jnp.float32)
        ranks = _dg(tri, oh, 1, 0) + running
        grank = jnp.sum(oh * (ranks - 1.0), axis=1, keepdims=True)
        pos_ref[pl.ds(c * RBLK, RBLK), :] = grank.astype(jnp.int32)
        return ranks[RBLK - 1:RBLK, :]

    counts = lax.fori_loop(0, RNB, pass1, jnp.zeros((1, E), jnp.float32))

    lt8_r = lax.broadcasted_iota(jnp.int32, (E, E), 0)
    lt8_c = lax.broadcasted_iota(jnp.int32, (E, E), 1)
    lt8 = (lt8_r < lt8_c).astype(jnp.float32)
    nblk = jnp.floor((counts + float(BLK - 1)) * (1.0 / BLK))
    ps = _dg(nblk, lt8, 1, 0)
    pad_off = ps * float(BLK)

    def pass2(c, carry):
        ids_c = ids_ref[pl.ds(c * RBLK, RBLK), :]
        oh = (ids_c == e_row).astype(jnp.float32)
        add = jnp.sum(oh * pad_off, axis=1, keepdims=True).astype(jnp.int32)
        pos_ref[pl.ds(c * RBLK, RBLK), :] = (
            pos_ref[pl.ds(c * RBLK, RBLK), :] + add)
        return carry

    lax.fori_loop(0, RNB, pass2, 0)

    total = ps[0, E - 1] + nblk[0, E - 1]
    s_f = lax.broadcasted_iota(jnp.int32, (SM, 1), 0).astype(jnp.float32)
    e_col = jnp.sum((ps <= s_f).astype(jnp.float32), axis=1,
                    keepdims=True) - 1.0
    e_col = jnp.clip(e_col, 0.0, float(E - 1))
    valid = (s_f < total).astype(jnp.float32)
    xb = jnp.minimum(s_f, total - 1.0)

    prev_e = jnp.concatenate(
        [jnp.full((1, 1), -1.0, jnp.float32), e_col[:SM - 1]], axis=0)
    chg = ((e_col != prev_e) & (valid > 0)).astype(jnp.float32)
    idm_r = lax.broadcasted_iota(jnp.int32, (SM, SM), 0)
    idm_c = lax.broadcasted_iota(jnp.int32, (SM, SM), 1)
    idm = (idm_r == idm_c).astype(jnp.float32)
    trim = (idm_r >= idm_c).astype(jnp.float32)
    dix = _dg(trim, chg, 1, 0) - 1.0
    slot = dix - 2.0 * jnp.floor(dix * 0.5)
    islot = 1.0 - slot
    dix_row = _dg(dix, idm, 0, 0)
    chg_row = _dg(chg, idm, 0, 0)
    nxt = (dix_row == dix + 1.0).astype(jnp.float32) * chg_row
    nxe = _dg(nxt, e_col, 1, 0)
    nxv = _dg(nxt, jnp.ones((SM, 1), jnp.float32), 1, 0)
    issue = chg * (nxv > 0).astype(jnp.float32)

    meta = jnp.concatenate(
        [xb, e_col, valid, chg, slot, issue, islot, nxe], axis=1)
    meta_ref[...] = meta.astype(jnp.int32)


def _route(ids):
    return pl.pallas_call(
        _routing_body,
        out_shape=(jax.ShapeDtypeStruct((T, 1), jnp.int32),
                   jax.ShapeDtypeStruct((SM, 8), jnp.int32)),
    )(ids)


@functools.lru_cache(maxsize=None)
def _sc_kernels():
    mesh = plsc.VectorSubcoreMesh(core_axis_name="c", subcore_axis_name="s")
    scratch = [pltpu.VMEM((CHUNK,), jnp.int32),
               pltpu.VMEM((CHUNK, D), jnp.float32),
               pltpu.SemaphoreType.DMA]

    @functools.partial(
        pl.kernel,
        out_type=jax.ShapeDtypeStruct((TP, D), jnp.float32),
        mesh=mesh, scratch_types=scratch)
    def dispatch(x_hbm, pos_hbm, out_hbm, idx_v, rows_v, sem):
        wid = lax.axis_index("s") * 2 + lax.axis_index("c")
        base = wid * CHUNK
        pltpu.sync_copy(pos_hbm.at[pl.ds(base, CHUNK)], idx_v)
        pltpu.sync_copy(x_hbm.at[pl.ds(base, CHUNK)], rows_v)
        pltpu.async_copy(rows_v, out_hbm.at[idx_v], sem).wait()

    @functools.partial(
        pl.kernel,
        out_type=jax.ShapeDtypeStruct((T, D), jnp.float32),
        mesh=mesh, scratch_types=scratch)
    def combine(y_hbm, pos_hbm, out_hbm, idx_v, rows_v, sem):
        wid = lax.axis_index("s") * 2 + lax.axis_index("c")
        base = wid * CHUNK
        pltpu.sync_copy(pos_hbm.at[pl.ds(base, CHUNK)], idx_v)
        pltpu.async_copy(y_hbm.at[idx_v], rows_v, sem).wait()
        pltpu.sync_copy(rows_v, out_hbm.at[pl.ds(base, CHUNK)])

    return dispatch, combine


def _mlp_body(meta_ref, x_ref, gw_hbm, uw_hbm, dw_hbm, o_ref,
              gw_v, uw_v, dw_v, sgw, suw, sdw):
    s = pl.program_id(0)
    ex = meta_ref[s, 1]
    va = meta_ref[s, 2]
    chg = meta_ref[s, 3]
    slot = meta_ref[s, 4]
    issue = meta_ref[s, 5]
    islot = meta_ref[s, 6]
    nxe = meta_ref[s, 7]

    def _start(e, k):
        pltpu.make_async_copy(gw_hbm.at[e], gw_v.at[k], sgw.at[k]).start()
        pltpu.make_async_copy(uw_hbm.at[e], uw_v.at[k], suw.at[k]).start()
        pltpu.make_async_copy(dw_hbm.at[e], dw_v.at[k], sdw.at[k]).start()

    @pl.when(s == 0)
    def _():
        _start(ex, slot)

    @pl.when(issue > 0)
    def _():
        _start(nxe, islot)

    @pl.when(va > 0)
    def _():
        x = x_ref[...]

        @pl.when(chg > 0)
        def _():
            pltpu.make_async_copy(gw_hbm.at[0], gw_v.at[slot],
                                  sgw.at[slot]).wait()

        g = _dg(x, gw_v[slot], 1, 1)

        @pl.when(chg > 0)
        def _():
            pltpu.make_async_copy(uw_hbm.at[0], uw_v.at[slot],
                                  suw.at[slot]).wait()

        u = _dg(x, uw_v[slot], 1, 1)
        h = (g * jax.nn.sigmoid(g)) * u

        @pl.when(chg > 0)
        def _():
            pltpu.make_async_copy(dw_hbm.at[0], dw_v.at[slot],
                                  sdw.at[slot]).wait()

        o_ref[...] = _dg(h, dw_v[slot], 1, 1)


def _mlp(meta, sorted_x, gate_w, up_w, down_w):
    grid_spec = pltpu.PrefetchScalarGridSpec(
        num_scalar_prefetch=1,
        grid=(SP,),
        in_specs=[
            pl.BlockSpec((BLK, D), lambda s, m: (m[s, 0], 0)),
            pl.BlockSpec(memory_space=pl.ANY),
            pl.BlockSpec(memory_space=pl.ANY),
            pl.BlockSpec(memory_space=pl.ANY),
        ],
        out_specs=pl.BlockSpec((BLK, D), lambda s, m: (m[s, 0], 0)),
        scratch_shapes=[
            pltpu.VMEM((2, FF, D), jnp.float32),
            pltpu.VMEM((2, FF, D), jnp.float32),
            pltpu.VMEM((2, D, FF), jnp.float32),
            pltpu.SemaphoreType.DMA((2,)),
            pltpu.SemaphoreType.DMA((2,)),
            pltpu.SemaphoreType.DMA((2,)),
        ],
    )
    return pl.pallas_call(
        _mlp_body,
        grid_spec=grid_spec,
        out_shape=jax.ShapeDtypeStruct((TP, D), jnp.float32),
    )(meta, sorted_x, gate_w, up_w, down_w)


def _scale_body(y_ref, w_ref, o_ref):
    o_ref[...] = y_ref[...] * w_ref[...]


def _scale(y, w):
    return pl.pallas_call(
        _scale_body,
        grid=(NB,),
        in_specs=[pl.BlockSpec((BLK, D), lambda i: (i, 0)),
                  pl.BlockSpec((BLK, 1), lambda i: (i, 0))],
        out_specs=pl.BlockSpec((BLK, D), lambda i: (i, 0)),
        out_shape=jax.ShapeDtypeStruct((T, D), jnp.float32),
    )(y, w)


def kernel(x, topk_ids, topk_weight, gate_w, up_w, down_w):
    ids = topk_ids.reshape(T, 1).astype(jnp.int32)
    pos2d, meta = _route(ids)
    pos = pos2d.reshape(T)

    dispatch, combine = _sc_kernels()
    sorted_x = dispatch(x, pos)
    sorted_y = _mlp(meta, sorted_x, gate_w, up_w, down_w)
    y_tok = combine(sorted_y, pos)
    return _scale(y_tok, topk_weight.reshape(T, 1).astype(jnp.float32))
